# Initial kernel scaffold; baseline (speedup 1.0000x reference)
import jax
import jax.numpy as jnp
from jax import lax
from jax.experimental import pallas as pl

N_POINTS = 4096
K_SELECT = 1024
D_OUT = 256
CLUSTER = 64
G_DIM = 256

ROW_BLK = 1024
N_BLKS = N_POINTS // ROW_BLK


def _attn_kernel(x_blk, x_full, xT, Wq, WkT, Wv, out_ref):
    G = jnp.dot(Wq[...], WkT[...], preferred_element_type=jnp.float32)
    q = jnp.dot(x_blk[...], G, preferred_element_type=jnp.float32)
    S = jnp.dot(q, xT[...], preferred_element_type=jnp.float32) / jnp.sqrt(
        jnp.float32(3.0)
    )
    rm = jnp.max(S, axis=1, keepdims=True)
    E = jnp.exp(S - rm)
    wvs = jnp.sum(Wv[...], axis=1, keepdims=True)
    vsum = jnp.dot(x_full[...], wvs, preferred_element_type=jnp.float32)
    v2 = jnp.concatenate([vsum, jnp.ones_like(vsum)], axis=1)
    r = jnp.dot(E, v2, preferred_element_type=jnp.float32)
    out_ref[...] = (r[:, 0] / r[:, 1])[:, None]


def _attn_scores(x2, xT, Wq, WkT, Wv):
    return pl.pallas_call(
        _attn_kernel,
        grid=(N_BLKS,),
        in_specs=[
            pl.BlockSpec((ROW_BLK, 3), lambda i: (i, 0)),
            pl.BlockSpec((N_POINTS, 3), lambda i: (0, 0)),
            pl.BlockSpec((3, N_POINTS), lambda i: (0, 0)),
            pl.BlockSpec((3, 3), lambda i: (0, 0)),
            pl.BlockSpec((3, 3), lambda i: (0, 0)),
            pl.BlockSpec((3, 3), lambda i: (0, 0)),
        ],
        out_specs=pl.BlockSpec((ROW_BLK, 1), lambda i: (i, 0)),
        out_shape=jax.ShapeDtypeStruct((N_POINTS, 1), jnp.float32),
    )(x2, xT, Wq, WkT, Wv)


def _mask_kernel(xg_ref, mask_ref):
    xg = xg_ref[...]
    b = lax.bitcast_convert_type(xg, jnp.int32)
    neg = jnp.bitwise_xor(jnp.bitwise_not(b), jnp.int32(-2147483648))
    key = jnp.where(b < 0, neg, b)

    kf = jnp.int32(K_SELECT)

    def body(_, carry):
        lo, hi = carry
        mid = (lo & hi) + ((lo ^ hi) >> 1)
        c = jnp.sum((key >= mid).astype(jnp.int32))
        ge = c >= kf
        return jnp.where(ge, mid, lo), jnp.where(ge, hi, mid)

    lo, hi = lax.fori_loop(
        0, 32, body, (jnp.int32(-2147483648), jnp.int32(2147483647))
    )
    T = lo
    gt = key > T
    eq = key == T
    need = kf - jnp.sum(gt.astype(jnp.int32))
    eqf = eq.astype(jnp.float32)
    li = lax.broadcasted_iota(jnp.int32, (128, 128), 0)
    lj = lax.broadcasted_iota(jnp.int32, (128, 128), 1)
    L = (li <= lj).astype(jnp.float32)
    cum_lane = jnp.dot(eqf, L, preferred_element_type=jnp.float32)
    row_tot = jnp.sum(eqf, axis=1, keepdims=True)
    ri = lax.broadcasted_iota(jnp.int32, (32, 32), 0)
    rj = lax.broadcasted_iota(jnp.int32, (32, 32), 1)
    M = (rj < ri).astype(jnp.float32)
    row_off = jnp.dot(M, row_tot, preferred_element_type=jnp.float32)
    cum = cum_lane + row_off
    sel_eq = jnp.logical_and(eq, cum <= need.astype(jnp.float32))
    mask_ref[...] = jnp.logical_or(gt, sel_eq).astype(jnp.float32)


def _topk_mask(xg_32x128):
    return pl.pallas_call(
        _mask_kernel,
        out_shape=jax.ShapeDtypeStruct((32, 128), jnp.float32),
    )(xg_32x128)


def _vlad_kernel(x_full, maskv, W1, b1, W2, b2, W3, b3, cw, g1, bt1, cw2,
                 u_ref):
    h = jnp.dot(x_full[...], W1[...], preferred_element_type=jnp.float32)
    h = jnp.maximum(h + b1[...], 0.0)
    h = jnp.dot(h, W2[...], preferred_element_type=jnp.float32)
    h = jnp.maximum(h + b2[...], 0.0)
    feat = jnp.dot(h, W3[...], preferred_element_type=jnp.float32) + b3[...]
    logits = jnp.dot(feat, cw[...], preferred_element_type=jnp.float32)
    logits = logits * g1[...] + bt1[...]
    mx = jnp.max(logits, axis=1, keepdims=True)
    ex = jnp.exp(logits - mx)
    act = ex / jnp.sum(ex, axis=1, keepdims=True)
    actm = act * maskv[...]
    a_sum = jnp.sum(actm, axis=0, keepdims=True)
    vlad = lax.dot_general(
        feat, actm, (((0,), (0,)), ((), ())),
        preferred_element_type=jnp.float32,
    )
    vlad = vlad - a_sum * cw2[...]
    nrm = jnp.sqrt(jnp.sum(vlad * vlad, axis=0, keepdims=True))
    u = vlad / (nrm + 1e-12)
    g = jnp.sqrt(jnp.sum(u * u))
    u_ref[...] = u / (g + 1e-12)


def _vlad(x2, maskv, W1, b1, W2, b2, W3, b3, cw, g1, bt1, cw2):
    return pl.pallas_call(
        _vlad_kernel,
        out_shape=jax.ShapeDtypeStruct((D_OUT, CLUSTER), jnp.float32),
    )(x2, maskv, W1, b1, W2, b2, W3, b3, cw, g1, bt1, cw2)


def _head_kernel(u_flat, hw, g2, b2, gw, gg, gb, out_ref):
    t = jnp.dot(u_flat[...], hw[...], preferred_element_type=jnp.float32)
    o = t * g2[...] + b2[...]
    z = jnp.dot(o, gw[...], preferred_element_type=jnp.float32)
    z = z * gg[...] + gb[...]
    gates = 1.0 / (1.0 + jnp.exp(-z))
    out_ref[...] = o * gates


def _head(u_flat, hw, g2, b2, gw, gg, gb):
    return pl.pallas_call(
        _head_kernel,
        out_shape=jax.ShapeDtypeStruct((1, G_DIM), jnp.float32),
    )(u_flat, hw, g2, b2, gw, gg, gb)


def kernel(x, Wq, Wk, Wv, W1, b1, W2, b2, W3, b3, cluster_w, bn1_gamma,
           bn1_beta, cluster_w2, hidden_w, bn2_gamma, bn2_beta, gating_w,
           gbn_gamma, gbn_beta):
    x2 = x[0]
    xT = x2.T
    xg = _attn_scores(x2, xT, Wq, Wk.T, Wv)
    x_gcn = xg.reshape(1, N_POINTS)

    maskf = _topk_mask(xg.reshape(32, 128))
    maskv = maskf.reshape(N_POINTS, 1)

    u = _vlad(
        x2, maskv, W1, b1.reshape(1, -1), W2, b2.reshape(1, -1), W3,
        b3.reshape(1, -1), cluster_w, bn1_gamma.reshape(1, -1),
        bn1_beta.reshape(1, -1), cluster_w2[0],
    )

    out = _head(
        u.reshape(1, D_OUT * CLUSTER), hidden_w, bn2_gamma.reshape(1, -1),
        bn2_beta.reshape(1, -1), gating_w, gbn_gamma.reshape(1, -1),
        gbn_beta.reshape(1, -1),
    )
    return out, x_gcn


# speedup vs baseline: 1.0138x; 1.0138x over previous
import jax
import jax.numpy as jnp
from jax import lax
from jax.experimental import pallas as pl

N_POINTS = 4096
K_SELECT = 1024
D_OUT = 256
CLUSTER = 64
G_DIM = 256

ROW_BLK = 1024
N_BLKS = N_POINTS // ROW_BLK


def _attn_kernel(x_blk, x_full, xT, Wq, WkT, Wv, out_ref):
    q = jnp.dot(x_blk[...], Wq[...], preferred_element_type=jnp.float32)
    kT = jnp.dot(WkT[...], xT[...], preferred_element_type=jnp.float32)
    v = jnp.dot(x_full[...], Wv[...], preferred_element_type=jnp.float32)
    S = jnp.dot(q, kT, preferred_element_type=jnp.float32) / jnp.sqrt(
        jnp.float32(3.0)
    )
    rm = jnp.max(S, axis=1, keepdims=True)
    E = jnp.exp(S - rm)
    attn = E / jnp.sum(E, axis=1, keepdims=True)
    x_att = jnp.dot(attn, v, preferred_element_type=jnp.float32)
    out_ref[...] = jnp.sum(x_att, axis=1, keepdims=True)


def _attn_scores(x2, xT, Wq, WkT, Wv):
    return pl.pallas_call(
        _attn_kernel,
        grid=(N_BLKS,),
        in_specs=[
            pl.BlockSpec((ROW_BLK, 3), lambda i: (i, 0)),
            pl.BlockSpec((N_POINTS, 3), lambda i: (0, 0)),
            pl.BlockSpec((3, N_POINTS), lambda i: (0, 0)),
            pl.BlockSpec((3, 3), lambda i: (0, 0)),
            pl.BlockSpec((3, 3), lambda i: (0, 0)),
            pl.BlockSpec((3, 3), lambda i: (0, 0)),
        ],
        out_specs=pl.BlockSpec((ROW_BLK, 1), lambda i: (i, 0)),
        out_shape=jax.ShapeDtypeStruct((N_POINTS, 1), jnp.float32),
    )(x2, x2, xT, Wq, WkT, Wv)


def _mask_kernel(xg_ref, mask_ref):
    xg = xg_ref[...]
    b = lax.bitcast_convert_type(xg, jnp.int32)
    neg = jnp.bitwise_xor(jnp.bitwise_not(b), jnp.int32(-2147483648))
    key = jnp.where(b < 0, neg, b)

    kf = jnp.int32(K_SELECT)

    def body(_, carry):
        lo, hi = carry
        mid = (lo & hi) + ((lo ^ hi) >> 1)
        c = jnp.sum((key >= mid).astype(jnp.int32))
        ge = c >= kf
        return jnp.where(ge, mid, lo), jnp.where(ge, hi, mid)

    lo, hi = lax.fori_loop(
        0, 32, body, (jnp.int32(-2147483648), jnp.int32(2147483647))
    )
    T = lo
    gt = key > T
    eq = key == T
    need = kf - jnp.sum(gt.astype(jnp.int32))
    eqf = eq.astype(jnp.float32)
    li = lax.broadcasted_iota(jnp.int32, (128, 128), 0)
    lj = lax.broadcasted_iota(jnp.int32, (128, 128), 1)
    L = (li <= lj).astype(jnp.float32)
    cum_lane = jnp.dot(eqf, L, preferred_element_type=jnp.float32)
    row_tot = jnp.sum(eqf, axis=1, keepdims=True)
    ri = lax.broadcasted_iota(jnp.int32, (32, 32), 0)
    rj = lax.broadcasted_iota(jnp.int32, (32, 32), 1)
    M = (rj < ri).astype(jnp.float32)
    row_off = jnp.dot(M, row_tot, preferred_element_type=jnp.float32)
    cum = cum_lane + row_off
    sel_eq = jnp.logical_and(eq, cum <= need.astype(jnp.float32))
    mask_ref[...] = jnp.logical_or(gt, sel_eq).astype(jnp.float32)


def _topk_mask(xg_32x128):
    return pl.pallas_call(
        _mask_kernel,
        out_shape=jax.ShapeDtypeStruct((32, 128), jnp.float32),
    )(xg_32x128)


def _vlad_kernel(x_full, maskv, W1, b1, W2, b2, W3, b3, cw, g1, bt1, cw2,
                 u_ref):
    h = jnp.dot(x_full[...], W1[...], preferred_element_type=jnp.float32)
    h = jnp.maximum(h + b1[...], 0.0)
    h = jnp.dot(h, W2[...], preferred_element_type=jnp.float32)
    h = jnp.maximum(h + b2[...], 0.0)
    feat = jnp.dot(h, W3[...], preferred_element_type=jnp.float32) + b3[...]
    logits = jnp.dot(feat, cw[...], preferred_element_type=jnp.float32)
    logits = logits * g1[...] + bt1[...]
    mx = jnp.max(logits, axis=1, keepdims=True)
    ex = jnp.exp(logits - mx)
    act = ex / jnp.sum(ex, axis=1, keepdims=True)
    actm = act * maskv[...]
    a_sum = jnp.sum(actm, axis=0, keepdims=True)
    vlad = lax.dot_general(
        feat, actm, (((0,), (0,)), ((), ())),
        preferred_element_type=jnp.float32,
    )
    vlad = vlad - a_sum * cw2[...]
    nrm = jnp.sqrt(jnp.sum(vlad * vlad, axis=0, keepdims=True))
    u = vlad / (nrm + 1e-12)
    g = jnp.sqrt(jnp.sum(u * u))
    u_ref[...] = u / (g + 1e-12)


def _vlad(x2, maskv, W1, b1, W2, b2, W3, b3, cw, g1, bt1, cw2):
    return pl.pallas_call(
        _vlad_kernel,
        out_shape=jax.ShapeDtypeStruct((D_OUT, CLUSTER), jnp.float32),
    )(x2, maskv, W1, b1, W2, b2, W3, b3, cw, g1, bt1, cw2)


def _head_kernel(u_flat, hw, g2, b2, gw, gg, gb, out_ref):
    t = jnp.dot(u_flat[...], hw[...], preferred_element_type=jnp.float32)
    o = t * g2[...] + b2[...]
    z = jnp.dot(o, gw[...], preferred_element_type=jnp.float32)
    z = z * gg[...] + gb[...]
    gates = 1.0 / (1.0 + jnp.exp(-z))
    out_ref[...] = o * gates


def _head(u_flat, hw, g2, b2, gw, gg, gb):
    return pl.pallas_call(
        _head_kernel,
        out_shape=jax.ShapeDtypeStruct((1, G_DIM), jnp.float32),
    )(u_flat, hw, g2, b2, gw, gg, gb)


def kernel(x, Wq, Wk, Wv, W1, b1, W2, b2, W3, b3, cluster_w, bn1_gamma,
           bn1_beta, cluster_w2, hidden_w, bn2_gamma, bn2_beta, gating_w,
           gbn_gamma, gbn_beta):
    x2 = x[0]
    xT = x2.T
    xg = _attn_scores(x2, xT, Wq, Wk.T, Wv)
    x_gcn = xg.reshape(1, N_POINTS)

    maskf = _topk_mask(xg.reshape(32, 128))
    maskv = maskf.reshape(N_POINTS, 1)

    u = _vlad(
        x2, maskv, W1, b1.reshape(1, -1), W2, b2.reshape(1, -1), W3,
        b3.reshape(1, -1), cluster_w, bn1_gamma.reshape(1, -1),
        bn1_beta.reshape(1, -1), cluster_w2[0],
    )

    out = _head(
        u.reshape(1, D_OUT * CLUSTER), hidden_w, bn2_gamma.reshape(1, -1),
        bn2_beta.reshape(1, -1), gating_w, gbn_gamma.reshape(1, -1),
        gbn_beta.reshape(1, -1),
    )
    return out, x_gcn


# speedup vs baseline: 1.1122x; 1.0971x over previous
import jax
import jax.numpy as jnp
from jax import lax
from jax.experimental import pallas as pl

N_POINTS = 4096
K_SELECT = 1024
D_OUT = 256
CLUSTER = 64
G_DIM = 256

ROW_BLK = 1024
N_BLKS = N_POINTS // ROW_BLK


def _attn_kernel(x_blk, x_full, xT, Wq, WkT, Wv, out_ref):
    q = jnp.dot(x_blk[...], Wq[...], preferred_element_type=jnp.float32)
    kT = jnp.dot(WkT[...], xT[...], preferred_element_type=jnp.float32)
    v = jnp.dot(x_full[...], Wv[...], preferred_element_type=jnp.float32)
    S = jnp.dot(q, kT, preferred_element_type=jnp.float32) / jnp.sqrt(
        jnp.float32(3.0)
    )
    rm = jnp.max(S, axis=1, keepdims=True)
    E = jnp.exp(S - rm)
    v4 = jnp.concatenate([v, jnp.ones_like(v[:, :1])], axis=1)
    r = jnp.dot(E, v4, preferred_element_type=jnp.float32)
    out_ref[...] = ((r[:, 0] + r[:, 1] + r[:, 2]) / r[:, 3])[:, None]


def _attn_scores(x2, xT, Wq, WkT, Wv):
    return pl.pallas_call(
        _attn_kernel,
        grid=(N_BLKS,),
        in_specs=[
            pl.BlockSpec((ROW_BLK, 3), lambda i: (i, 0)),
            pl.BlockSpec((N_POINTS, 3), lambda i: (0, 0)),
            pl.BlockSpec((3, N_POINTS), lambda i: (0, 0)),
            pl.BlockSpec((3, 3), lambda i: (0, 0)),
            pl.BlockSpec((3, 3), lambda i: (0, 0)),
            pl.BlockSpec((3, 3), lambda i: (0, 0)),
        ],
        out_specs=pl.BlockSpec((ROW_BLK, 1), lambda i: (i, 0)),
        out_shape=jax.ShapeDtypeStruct((N_POINTS, 1), jnp.float32),
    )(x2, x2, xT, Wq, WkT, Wv)


def _mask_kernel(xg_ref, mask_ref):
    xg = xg_ref[...]
    b = lax.bitcast_convert_type(xg, jnp.int32)
    neg = jnp.bitwise_xor(jnp.bitwise_not(b), jnp.int32(-2147483648))
    key = jnp.where(b < 0, neg, b)

    kf = jnp.int32(K_SELECT)

    def body(_, carry):
        lo, hi = carry
        mid = (lo & hi) + ((lo ^ hi) >> 1)
        c = jnp.sum((key >= mid).astype(jnp.int32))
        ge = c >= kf
        return jnp.where(ge, mid, lo), jnp.where(ge, hi, mid)

    lo, hi = lax.fori_loop(
        0, 32, body, (jnp.int32(-2147483648), jnp.int32(2147483647))
    )
    T = lo
    gt = key > T
    eq = key == T
    need = kf - jnp.sum(gt.astype(jnp.int32))
    eqf = eq.astype(jnp.float32)
    li = lax.broadcasted_iota(jnp.int32, (128, 128), 0)
    lj = lax.broadcasted_iota(jnp.int32, (128, 128), 1)
    L = (li <= lj).astype(jnp.float32)
    cum_lane = jnp.dot(eqf, L, preferred_element_type=jnp.float32)
    row_tot = jnp.sum(eqf, axis=1, keepdims=True)
    ri = lax.broadcasted_iota(jnp.int32, (32, 32), 0)
    rj = lax.broadcasted_iota(jnp.int32, (32, 32), 1)
    M = (rj < ri).astype(jnp.float32)
    row_off = jnp.dot(M, row_tot, preferred_element_type=jnp.float32)
    cum = cum_lane + row_off
    sel_eq = jnp.logical_and(eq, cum <= need.astype(jnp.float32))
    mask_ref[...] = jnp.logical_or(gt, sel_eq).astype(jnp.float32)


def _topk_mask(xg_32x128):
    return pl.pallas_call(
        _mask_kernel,
        out_shape=jax.ShapeDtypeStruct((32, 128), jnp.float32),
    )(xg_32x128)


def _vlad_kernel(x_full, maskv, W1, b1, W2, b2, W3, b3, cw, g1, bt1, cw2,
                 u_ref):
    h = jnp.dot(x_full[...], W1[...], preferred_element_type=jnp.float32)
    h = jnp.maximum(h + b1[...], 0.0)
    h = jnp.dot(h, W2[...], preferred_element_type=jnp.float32)
    h = jnp.maximum(h + b2[...], 0.0)
    feat = jnp.dot(h, W3[...], preferred_element_type=jnp.float32) + b3[...]
    logits = jnp.dot(feat, cw[...], preferred_element_type=jnp.float32)
    logits = logits * g1[...] + bt1[...]
    mx = jnp.max(logits, axis=1, keepdims=True)
    ex = jnp.exp(logits - mx)
    act = ex / jnp.sum(ex, axis=1, keepdims=True)
    actm = act * maskv[...]
    a_sum = jnp.sum(actm, axis=0, keepdims=True)
    vlad = lax.dot_general(
        feat, actm, (((0,), (0,)), ((), ())),
        preferred_element_type=jnp.float32,
    )
    vlad = vlad - a_sum * cw2[...]
    nrm = jnp.sqrt(jnp.sum(vlad * vlad, axis=0, keepdims=True))
    u = vlad / (nrm + 1e-12)
    g = jnp.sqrt(jnp.sum(u * u))
    u_ref[...] = u / (g + 1e-12)


def _vlad(x2, maskv, W1, b1, W2, b2, W3, b3, cw, g1, bt1, cw2):
    return pl.pallas_call(
        _vlad_kernel,
        out_shape=jax.ShapeDtypeStruct((D_OUT, CLUSTER), jnp.float32),
    )(x2, maskv, W1, b1, W2, b2, W3, b3, cw, g1, bt1, cw2)


def _head_kernel(u_flat, hw, g2, b2, gw, gg, gb, out_ref):
    t = jnp.dot(u_flat[...], hw[...], preferred_element_type=jnp.float32)
    o = t * g2[...] + b2[...]
    z = jnp.dot(o, gw[...], preferred_element_type=jnp.float32)
    z = z * gg[...] + gb[...]
    gates = 1.0 / (1.0 + jnp.exp(-z))
    out_ref[...] = o * gates


def _head(u_flat, hw, g2, b2, gw, gg, gb):
    return pl.pallas_call(
        _head_kernel,
        out_shape=jax.ShapeDtypeStruct((1, G_DIM), jnp.float32),
    )(u_flat, hw, g2, b2, gw, gg, gb)


def kernel(x, Wq, Wk, Wv, W1, b1, W2, b2, W3, b3, cluster_w, bn1_gamma,
           bn1_beta, cluster_w2, hidden_w, bn2_gamma, bn2_beta, gating_w,
           gbn_gamma, gbn_beta):
    x2 = x[0]
    xT = x2.T
    xg = _attn_scores(x2, xT, Wq, Wk.T, Wv)
    x_gcn = xg.reshape(1, N_POINTS)

    maskf = _topk_mask(xg.reshape(32, 128))
    maskv = maskf.reshape(N_POINTS, 1)

    u = _vlad(
        x2, maskv, W1, b1.reshape(1, -1), W2, b2.reshape(1, -1), W3,
        b3.reshape(1, -1), cluster_w, bn1_gamma.reshape(1, -1),
        bn1_beta.reshape(1, -1), cluster_w2[0],
    )

    out = _head(
        u.reshape(1, D_OUT * CLUSTER), hidden_w, bn2_gamma.reshape(1, -1),
        bn2_beta.reshape(1, -1), gating_w, gbn_gamma.reshape(1, -1),
        gbn_beta.reshape(1, -1),
    )
    return out, x_gcn


# speedup vs baseline: 1.4320x; 1.2876x over previous
import jax
import jax.numpy as jnp
from jax import lax
from jax.experimental import pallas as pl

N_POINTS = 4096
K_SELECT = 1024
D_OUT = 256
CLUSTER = 64
G_DIM = 256

ROW_BLK = 1024
N_BLKS = N_POINTS // ROW_BLK
HID_BLK = 2048


def _main_kernel(x2, xT, Wq, WkT, Wv, W1, b1, W2, b2, W3, b3, cw, g1, bt1,
                 cw2, xg_ref, u_ref, maskv_ref):
    kT = jnp.dot(WkT[...], xT[...], preferred_element_type=jnp.float32)
    v = jnp.dot(x2[...], Wv[...], preferred_element_type=jnp.float32)
    v4 = jnp.concatenate([v, jnp.ones_like(v[:, :1])], axis=1)
    for b in range(N_BLKS):
        x_blk = x2[pl.ds(b * ROW_BLK, ROW_BLK), :]
        q = jnp.dot(x_blk, Wq[...], preferred_element_type=jnp.float32)
        S = jnp.dot(q, kT, preferred_element_type=jnp.float32) / jnp.sqrt(
            jnp.float32(3.0)
        )
        rm = jnp.max(S, axis=1, keepdims=True)
        E = jnp.exp(S - rm)
        r = jnp.dot(E, v4, preferred_element_type=jnp.float32)
        xgb = (r[:, 0] + r[:, 1] + r[:, 2]) / r[:, 3]
        xg_ref[pl.ds(b, 1), :] = xgb.reshape(1, ROW_BLK)

    xg = xg_ref[...]
    bb = lax.bitcast_convert_type(xg, jnp.int32)
    negk = jnp.bitwise_xor(jnp.bitwise_not(bb), jnp.int32(-2147483648))
    key = jnp.where(bb < 0, negk, bb)
    kf = jnp.int32(K_SELECT)

    def tbody(_, carry):
        lo, hi = carry
        mid = (lo & hi) + ((lo ^ hi) >> 1)
        ge = jnp.sum((key >= mid).astype(jnp.int32)) >= kf
        return jnp.where(ge, mid, lo), jnp.where(ge, hi, mid)

    T, _ = lax.fori_loop(
        0, 32, tbody, (jnp.int32(-2147483648), jnp.int32(2147483647))
    )
    gt = key > T
    eq = key == T
    n_gt = jnp.sum(gt.astype(jnp.int32))
    idx = (lax.broadcasted_iota(jnp.int32, (N_BLKS, ROW_BLK), 0) * ROW_BLK
           + lax.broadcasted_iota(jnp.int32, (N_BLKS, ROW_BLK), 1))

    def jbody(_, carry):
        lo, hi = carry
        mid = (lo + hi) >> 1
        c = n_gt + jnp.sum((eq & (idx <= mid)).astype(jnp.int32))
        ge = c >= kf
        return jnp.where(ge, lo, mid + 1), jnp.where(ge, mid, hi)

    _, J = lax.fori_loop(
        0, 13, jbody, (jnp.int32(0), jnp.int32(N_POINTS - 1))
    )
    mask = jnp.logical_or(gt, jnp.logical_and(eq, idx <= J))
    maskf = mask.astype(jnp.float32)
    for b in range(N_BLKS):
        maskv_ref[pl.ds(b * ROW_BLK, ROW_BLK), :] = maskf[
            b:b + 1, :
        ].reshape(ROW_BLK, 1)

    h = jnp.dot(x2[...], W1[...], preferred_element_type=jnp.float32)
    h = jnp.maximum(h + b1[...], 0.0)
    h = jnp.dot(h, W2[...], preferred_element_type=jnp.float32)
    h = jnp.maximum(h + b2[...], 0.0)
    feat = jnp.dot(h, W3[...], preferred_element_type=jnp.float32) + b3[...]
    logits = jnp.dot(feat, cw[...], preferred_element_type=jnp.float32)
    logits = logits * g1[...] + bt1[...]
    mx = jnp.max(logits, axis=1, keepdims=True)
    ex = jnp.exp(logits - mx)
    act = ex / jnp.sum(ex, axis=1, keepdims=True)
    actm = act * maskv_ref[...]
    a_sum = jnp.sum(actm, axis=0, keepdims=True)
    vlad = lax.dot_general(
        feat, actm, (((0,), (0,)), ((), ())),
        preferred_element_type=jnp.float32,
    )
    vlad = vlad - a_sum * cw2[...]
    nrm = jnp.sqrt(jnp.sum(vlad * vlad, axis=0, keepdims=True))
    u = vlad / (nrm + 1e-12)
    g = jnp.sqrt(jnp.sum(u * u))
    u_ref[...] = u / (g + 1e-12)


def _main(x2, xT, Wq, WkT, Wv, W1, b1, W2, b2, W3, b3, cw, g1, bt1, cw2):
    return pl.pallas_call(
        _main_kernel,
        out_shape=(
            jax.ShapeDtypeStruct((N_BLKS, ROW_BLK), jnp.float32),
            jax.ShapeDtypeStruct((D_OUT, CLUSTER), jnp.float32),
            jax.ShapeDtypeStruct((N_POINTS, 1), jnp.float32),
        ),
    )(x2, xT, Wq, WkT, Wv, W1, b1, W2, b2, W3, b3, cw, g1, bt1, cw2)


def _head_kernel(u_flat, hw, g2, b2, gw, gg, gb, out_ref, acc):
    i = pl.program_id(0)
    t = jnp.dot(u_flat[...], hw[...], preferred_element_type=jnp.float32)

    @pl.when(i == 0)
    def _():
        acc[...] = t

    @pl.when(i > 0)
    def _():
        acc[...] = acc[...] + t

    @pl.when(i == (D_OUT * CLUSTER // HID_BLK) - 1)
    def _():
        o = acc[...] * g2[...] + b2[...]
        z = jnp.dot(o, gw[...], preferred_element_type=jnp.float32)
        z = z * gg[...] + gb[...]
        gates = 1.0 / (1.0 + jnp.exp(-z))
        out_ref[...] = o * gates


def _head(u_flat, hw, g2, b2, gw, gg, gb):
    from jax.experimental.pallas import tpu as pltpu

    nsteps = D_OUT * CLUSTER // HID_BLK
    return pl.pallas_call(
        _head_kernel,
        grid=(nsteps,),
        in_specs=[
            pl.BlockSpec((1, HID_BLK), lambda i: (0, i)),
            pl.BlockSpec((HID_BLK, G_DIM), lambda i: (i, 0)),
            pl.BlockSpec((1, G_DIM), lambda i: (0, 0)),
            pl.BlockSpec((1, G_DIM), lambda i: (0, 0)),
            pl.BlockSpec((G_DIM, G_DIM), lambda i: (0, 0)),
            pl.BlockSpec((1, G_DIM), lambda i: (0, 0)),
            pl.BlockSpec((1, G_DIM), lambda i: (0, 0)),
        ],
        out_specs=pl.BlockSpec((1, G_DIM), lambda i: (0, 0)),
        out_shape=jax.ShapeDtypeStruct((1, G_DIM), jnp.float32),
        scratch_shapes=[pltpu.VMEM((1, G_DIM), jnp.float32)],
    )(u_flat, hw, g2, b2, gw, gg, gb)


def kernel(x, Wq, Wk, Wv, W1, b1, W2, b2, W3, b3, cluster_w, bn1_gamma,
           bn1_beta, cluster_w2, hidden_w, bn2_gamma, bn2_beta, gating_w,
           gbn_gamma, gbn_beta):
    x2 = x[0]
    xT = x2.T
    xg, u, _ = _main(
        x2, xT, Wq, Wk.T, Wv, W1, b1.reshape(1, -1), W2, b2.reshape(1, -1),
        W3, b3.reshape(1, -1), cluster_w, bn1_gamma.reshape(1, -1),
        bn1_beta.reshape(1, -1), cluster_w2[0],
    )
    x_gcn = xg.reshape(1, N_POINTS)

    out = _head(
        u.reshape(1, D_OUT * CLUSTER), hidden_w, bn2_gamma.reshape(1, -1),
        bn2_beta.reshape(1, -1), gating_w, gbn_gamma.reshape(1, -1),
        gbn_beta.reshape(1, -1),
    )
    return out, x_gcn


# speedup vs baseline: 1.7141x; 1.1970x over previous
import jax
import jax.numpy as jnp
from jax import lax
from jax.experimental import pallas as pl
from jax.experimental.pallas import tpu as pltpu

N_POINTS = 4096
K_SELECT = 1024
D_OUT = 256
CLUSTER = 64
G_DIM = 256

ROW_BLK = 512
N_BLKS = N_POINTS // ROW_BLK


def _fused_kernel(x2, xT, Wq, WkT, Wv, W1, b1, W2, b2, W3, b3, cw, g1, bt1,
                  cw2, hw_hbm, g2, b2h, gw, gg, gb,
                  xg_ref, out_ref,
                  maskv_ref, uf_ref, hw_vmem, sem):
    cp = pltpu.make_async_copy(hw_hbm, hw_vmem, sem)
    cp.start()

    kT = jnp.dot(WkT[...], xT[...], preferred_element_type=jnp.float32)
    v = jnp.dot(x2[...], Wv[...], preferred_element_type=jnp.float32)
    v4 = jnp.concatenate([v, jnp.ones_like(v[:, :1])], axis=1)
    for b in range(N_BLKS):
        x_blk = x2[pl.ds(b * ROW_BLK, ROW_BLK), :]
        q = jnp.dot(x_blk, Wq[...], preferred_element_type=jnp.float32)
        S = jnp.dot(q, kT, preferred_element_type=jnp.float32) / jnp.sqrt(
            jnp.float32(3.0)
        )
        rm = jnp.max(S, axis=1, keepdims=True)
        E = jnp.exp(S - rm)
        r = jnp.dot(E, v4, preferred_element_type=jnp.float32)
        xgb = (r[:, 0] + r[:, 1] + r[:, 2]) / r[:, 3]
        xg_ref[pl.ds(b, 1), :] = xgb.reshape(1, ROW_BLK)

    xg = xg_ref[...]
    bb = lax.bitcast_convert_type(xg, jnp.int32)
    negk = jnp.bitwise_xor(jnp.bitwise_not(bb), jnp.int32(-2147483648))
    key = jnp.where(bb < 0, negk, bb)
    kf = jnp.int32(K_SELECT)

    def tbody(_, carry):
        lo, hi = carry
        mid = (lo & hi) + ((lo ^ hi) >> 1)
        ge = jnp.sum((key >= mid).astype(jnp.int32)) >= kf
        return jnp.where(ge, mid, lo), jnp.where(ge, hi, mid)

    T, _ = lax.fori_loop(
        0, 32, tbody, (jnp.int32(-2147483648), jnp.int32(2147483647))
    )
    gt = key > T
    eq = key == T
    n_gt = jnp.sum(gt.astype(jnp.int32))
    idx = (lax.broadcasted_iota(jnp.int32, (N_BLKS, ROW_BLK), 0) * ROW_BLK
           + lax.broadcasted_iota(jnp.int32, (N_BLKS, ROW_BLK), 1))

    def jbody(_, carry):
        lo, hi = carry
        mid = (lo + hi) >> 1
        c = n_gt + jnp.sum((eq & (idx <= mid)).astype(jnp.int32))
        ge = c >= kf
        return jnp.where(ge, lo, mid + 1), jnp.where(ge, mid, hi)

    _, J = lax.fori_loop(
        0, 13, jbody, (jnp.int32(0), jnp.int32(N_POINTS - 1))
    )
    mask = jnp.logical_or(gt, jnp.logical_and(eq, idx <= J))
    maskf = mask.astype(jnp.float32)
    for b in range(N_BLKS):
        maskv_ref[pl.ds(b * ROW_BLK, ROW_BLK), :] = maskf[
            b:b + 1, :
        ].reshape(ROW_BLK, 1)

    h = jnp.dot(x2[...], W1[...], preferred_element_type=jnp.float32)
    h = jnp.maximum(h + b1[...], 0.0)
    h = jnp.dot(h, W2[...], preferred_element_type=jnp.float32)
    h = jnp.maximum(h + b2[...], 0.0)
    feat = jnp.dot(h, W3[...], preferred_element_type=jnp.float32) + b3[...]
    logits = jnp.dot(feat, cw[...], preferred_element_type=jnp.float32)
    logits = logits * g1[...] + bt1[...]
    mx = jnp.max(logits, axis=1, keepdims=True)
    ex = jnp.exp(logits - mx)
    act = ex / jnp.sum(ex, axis=1, keepdims=True)
    actm = act * maskv_ref[...]
    a_sum = jnp.sum(actm, axis=0, keepdims=True)
    vlad = lax.dot_general(
        feat, actm, (((0,), (0,)), ((), ())),
        preferred_element_type=jnp.float32,
    )
    vlad = vlad - a_sum * cw2[...]
    nrm = jnp.sqrt(jnp.sum(vlad * vlad, axis=0, keepdims=True))
    u = vlad / (nrm + 1e-12)
    g = jnp.sqrt(jnp.sum(u * u))
    u = u / (g + 1e-12)

    for d in range(D_OUT):
        uf_ref[0:1, d * CLUSTER:(d + 1) * CLUSTER] = u[d:d + 1, :]
    cp.wait()
    t = jnp.dot(uf_ref[...], hw_vmem[...], preferred_element_type=jnp.float32)
    o = t * g2[...] + b2h[...]
    z = jnp.dot(o, gw[...], preferred_element_type=jnp.float32)
    z = z * gg[...] + gb[...]
    gates = 1.0 / (1.0 + jnp.exp(-z))
    out_ref[...] = o * gates


def _fused(x2, xT, Wq, WkT, Wv, W1, b1, W2, b2, W3, b3, cw, g1, bt1, cw2,
           hw, g2, b2h, gw, gg, gb):
    n_in = 21
    specs = [pl.BlockSpec(memory_space=pl.ANY) if i == 15
             else pl.BlockSpec() for i in range(n_in)]
    return pl.pallas_call(
        _fused_kernel,
        in_specs=specs,
        out_shape=(
            jax.ShapeDtypeStruct((N_BLKS, ROW_BLK), jnp.float32),
            jax.ShapeDtypeStruct((1, G_DIM), jnp.float32),
        ),
        scratch_shapes=[
            pltpu.VMEM((N_POINTS, 1), jnp.float32),
            pltpu.VMEM((1, D_OUT * CLUSTER), jnp.float32),
            pltpu.VMEM((D_OUT * CLUSTER, G_DIM), jnp.float32),
            pltpu.SemaphoreType.DMA,
        ],
    )(x2, xT, Wq, WkT, Wv, W1, b1, W2, b2, W3, b3, cw, g1, bt1, cw2,
      hw, g2, b2h, gw, gg, gb)


def kernel(x, Wq, Wk, Wv, W1, b1, W2, b2, W3, b3, cluster_w, bn1_gamma,
           bn1_beta, cluster_w2, hidden_w, bn2_gamma, bn2_beta, gating_w,
           gbn_gamma, gbn_beta):
    x2 = x[0]
    xT = x2.T
    xg, out = _fused(
        x2, xT, Wq, Wk.T, Wv, W1, b1.reshape(1, -1), W2, b2.reshape(1, -1),
        W3, b3.reshape(1, -1), cluster_w, bn1_gamma.reshape(1, -1),
        bn1_beta.reshape(1, -1), cluster_w2[0], hidden_w,
        bn2_gamma.reshape(1, -1), bn2_beta.reshape(1, -1), gating_w,
        gbn_gamma.reshape(1, -1), gbn_beta.reshape(1, -1),
    )
    return out, xg.reshape(1, N_POINTS)


# speedup vs baseline: 1.8093x; 1.0556x over previous
import jax
import jax.numpy as jnp
from jax import lax
from jax.experimental import pallas as pl
from jax.experimental.pallas import tpu as pltpu

N_POINTS = 4096
K_SELECT = 1024
D_OUT = 256
CLUSTER = 64
G_DIM = 256

ROW_BLK = 512
N_BLKS = N_POINTS // ROW_BLK


def _fused_kernel(x2, xT, Wq, WkT, Wv, W1, b1, W2, b2, W3, b3, cw, g1, bt1,
                  cw2, hw_hbm, g2, b2h, gw, gg, gb,
                  xg_ref, out_ref,
                  maskv_ref, uf_ref, hw_vmem, sem):
    cp = pltpu.make_async_copy(hw_hbm, hw_vmem, sem)
    cp.start()

    kT = jnp.dot(WkT[...], xT[...], preferred_element_type=jnp.float32)
    v = jnp.dot(x2[...], Wv[...], preferred_element_type=jnp.float32)
    v4 = jnp.concatenate(
        [v, jnp.ones_like(v[:, :1])], axis=1
    ).astype(jnp.bfloat16)
    rs3 = 1.0 / jnp.sqrt(jnp.float32(3.0))
    for b in range(N_BLKS):
        x_blk = x2[pl.ds(b * ROW_BLK, ROW_BLK), :]
        q = jnp.dot(x_blk, Wq[...], preferred_element_type=jnp.float32)
        S = jnp.dot(q, kT, preferred_element_type=jnp.float32)
        rm = jnp.max(S, axis=1, keepdims=True)
        E = jnp.exp((S - rm) * rs3).astype(jnp.bfloat16)
        r = jnp.dot(E, v4, preferred_element_type=jnp.float32)
        xgb = (r[:, 0] + r[:, 1] + r[:, 2]) / r[:, 3]
        xg_ref[pl.ds(b, 1), :] = xgb.reshape(1, ROW_BLK)

    xg = xg_ref[...]
    bb = lax.bitcast_convert_type(xg, jnp.int32)
    negk = jnp.bitwise_xor(jnp.bitwise_not(bb), jnp.int32(-2147483648))
    key = jnp.where(bb < 0, negk, bb)
    kf = jnp.int32(K_SELECT)

    def tbody(_, carry):
        lo, hi = carry
        mid = (lo & hi) + ((lo ^ hi) >> 1)
        ge = jnp.sum((key >= mid).astype(jnp.int32)) >= kf
        return jnp.where(ge, mid, lo), jnp.where(ge, hi, mid)

    T, _ = lax.fori_loop(
        0, 32, tbody, (jnp.int32(-2147483648), jnp.int32(2147483647))
    )
    gt = key > T
    eq = key == T
    n_gt = jnp.sum(gt.astype(jnp.int32))
    idx = (lax.broadcasted_iota(jnp.int32, (N_BLKS, ROW_BLK), 0) * ROW_BLK
           + lax.broadcasted_iota(jnp.int32, (N_BLKS, ROW_BLK), 1))

    def jbody(_, carry):
        lo, hi = carry
        mid = (lo + hi) >> 1
        c = n_gt + jnp.sum((eq & (idx <= mid)).astype(jnp.int32))
        ge = c >= kf
        return jnp.where(ge, lo, mid + 1), jnp.where(ge, mid, hi)

    _, J = lax.fori_loop(
        0, 13, jbody, (jnp.int32(0), jnp.int32(N_POINTS - 1))
    )
    mask = jnp.logical_or(gt, jnp.logical_and(eq, idx <= J))
    maskf = mask.astype(jnp.float32)
    for b in range(N_BLKS):
        maskv_ref[pl.ds(b * ROW_BLK, ROW_BLK), :] = maskf[
            b:b + 1, :
        ].reshape(ROW_BLK, 1)

    h = jnp.dot(x2[...], W1[...], preferred_element_type=jnp.float32)
    h = jnp.maximum(h + b1[...], 0.0)
    h = jnp.dot(h, W2[...], preferred_element_type=jnp.float32)
    h = jnp.maximum(h + b2[...], 0.0)
    feat = jnp.dot(h, W3[...], preferred_element_type=jnp.float32) + b3[...]
    logits = jnp.dot(feat, cw[...], preferred_element_type=jnp.float32)
    logits = logits * g1[...] + bt1[...]
    mx = jnp.max(logits, axis=1, keepdims=True)
    ex = jnp.exp(logits - mx)
    act = ex / jnp.sum(ex, axis=1, keepdims=True)
    actm = act * maskv_ref[...]
    a_sum = jnp.sum(actm, axis=0, keepdims=True)
    vlad = lax.dot_general(
        feat, actm, (((0,), (0,)), ((), ())),
        preferred_element_type=jnp.float32,
    )
    vlad = vlad - a_sum * cw2[...]
    nrm = jnp.sqrt(jnp.sum(vlad * vlad, axis=0, keepdims=True))
    u = vlad / (nrm + 1e-12)
    g = jnp.sqrt(jnp.sum(u * u))
    u = u / (g + 1e-12)

    for d in range(D_OUT):
        uf_ref[0:1, d * CLUSTER:(d + 1) * CLUSTER] = u[d:d + 1, :]
    cp.wait()
    t = jnp.dot(uf_ref[...], hw_vmem[...], preferred_element_type=jnp.float32)
    o = t * g2[...] + b2h[...]
    z = jnp.dot(o, gw[...], preferred_element_type=jnp.float32)
    z = z * gg[...] + gb[...]
    gates = 1.0 / (1.0 + jnp.exp(-z))
    out_ref[...] = o * gates


def _fused(x2, xT, Wq, WkT, Wv, W1, b1, W2, b2, W3, b3, cw, g1, bt1, cw2,
           hw, g2, b2h, gw, gg, gb):
    n_in = 21
    specs = [pl.BlockSpec(memory_space=pl.ANY) if i == 15
             else pl.BlockSpec() for i in range(n_in)]
    return pl.pallas_call(
        _fused_kernel,
        in_specs=specs,
        out_shape=(
            jax.ShapeDtypeStruct((N_BLKS, ROW_BLK), jnp.float32),
            jax.ShapeDtypeStruct((1, G_DIM), jnp.float32),
        ),
        scratch_shapes=[
            pltpu.VMEM((N_POINTS, 1), jnp.float32),
            pltpu.VMEM((1, D_OUT * CLUSTER), jnp.float32),
            pltpu.VMEM((D_OUT * CLUSTER, G_DIM), jnp.float32),
            pltpu.SemaphoreType.DMA,
        ],
    )(x2, xT, Wq, WkT, Wv, W1, b1, W2, b2, W3, b3, cw, g1, bt1, cw2,
      hw, g2, b2h, gw, gg, gb)


def kernel(x, Wq, Wk, Wv, W1, b1, W2, b2, W3, b3, cluster_w, bn1_gamma,
           bn1_beta, cluster_w2, hidden_w, bn2_gamma, bn2_beta, gating_w,
           gbn_gamma, gbn_beta):
    x2 = x[0]
    xT = x2.T
    xg, out = _fused(
        x2, xT, Wq, Wk.T, Wv, W1, b1.reshape(1, -1), W2, b2.reshape(1, -1),
        W3, b3.reshape(1, -1), cluster_w, bn1_gamma.reshape(1, -1),
        bn1_beta.reshape(1, -1), cluster_w2[0], hidden_w,
        bn2_gamma.reshape(1, -1), bn2_beta.reshape(1, -1), gating_w,
        gbn_gamma.reshape(1, -1), gbn_beta.reshape(1, -1),
    )
    return out, xg.reshape(1, N_POINTS)


# speedup vs baseline: 1.8497x; 1.0223x over previous
import jax
import jax.numpy as jnp
from jax import lax
from jax.experimental import pallas as pl
from jax.experimental.pallas import tpu as pltpu

N_POINTS = 4096
K_SELECT = 1024
D_OUT = 256
CLUSTER = 64
G_DIM = 256

ROW_BLK = 512
N_BLKS = N_POINTS // ROW_BLK


def _fused_kernel(x2, Wq, Wk, Wv, W1, b1, W2, b2, W3, b3, cw, g1, bt1,
                  cw2, hw_hbm, g2, b2h, gw, gg, gb,
                  xg_ref, out_ref,
                  maskv_ref, uf_ref, hw_vmem, sem):
    cp = pltpu.make_async_copy(hw_hbm, hw_vmem, sem)
    cp.start()

    kT = jnp.transpose(
        jnp.dot(x2[...], Wk[...], preferred_element_type=jnp.float32)
    )
    v = jnp.dot(x2[...], Wv[...], preferred_element_type=jnp.float32)
    v4 = jnp.concatenate(
        [v, jnp.ones_like(v[:, :1])], axis=1
    ).astype(jnp.bfloat16)
    rs3 = 1.0 / jnp.sqrt(jnp.float32(3.0))
    for b in range(N_BLKS):
        x_blk = x2[pl.ds(b * ROW_BLK, ROW_BLK), :]
        q = jnp.dot(x_blk, Wq[...], preferred_element_type=jnp.float32)
        S = jnp.dot(q, kT, preferred_element_type=jnp.float32)
        rm = jnp.max(S, axis=1, keepdims=True)
        E = jnp.exp((S - rm) * rs3).astype(jnp.bfloat16)
        r = jnp.dot(E, v4, preferred_element_type=jnp.float32)
        xgb = (r[:, 0] + r[:, 1] + r[:, 2]) / r[:, 3]
        xg_ref[pl.ds(b, 1), :] = xgb.reshape(1, ROW_BLK)

    xg = xg_ref[...]
    bb = lax.bitcast_convert_type(xg, jnp.int32)
    negk = jnp.bitwise_xor(jnp.bitwise_not(bb), jnp.int32(-2147483648))
    key = jnp.where(bb < 0, negk, bb)
    kf = jnp.int32(K_SELECT)

    def tbody(_, carry):
        lo, hi = carry
        mid = (lo & hi) + ((lo ^ hi) >> 1)
        ge = jnp.sum((key >= mid).astype(jnp.int32)) >= kf
        return jnp.where(ge, mid, lo), jnp.where(ge, hi, mid)

    T, _ = lax.fori_loop(
        0, 32, tbody, (jnp.int32(-2147483648), jnp.int32(2147483647))
    )
    gt = key > T
    eq = key == T
    n_gt = jnp.sum(gt.astype(jnp.int32))
    idx = (lax.broadcasted_iota(jnp.int32, (N_BLKS, ROW_BLK), 0) * ROW_BLK
           + lax.broadcasted_iota(jnp.int32, (N_BLKS, ROW_BLK), 1))

    def jbody(_, carry):
        lo, hi = carry
        mid = (lo + hi) >> 1
        c = n_gt + jnp.sum((eq & (idx <= mid)).astype(jnp.int32))
        ge = c >= kf
        return jnp.where(ge, lo, mid + 1), jnp.where(ge, mid, hi)

    _, J = lax.fori_loop(
        0, 13, jbody, (jnp.int32(0), jnp.int32(N_POINTS - 1))
    )
    mask = jnp.logical_or(gt, jnp.logical_and(eq, idx <= J))
    maskf = mask.astype(jnp.float32)
    for b in range(N_BLKS):
        maskv_ref[pl.ds(b * ROW_BLK, ROW_BLK), :] = maskf[
            b:b + 1, :
        ].reshape(ROW_BLK, 1)

    h = jnp.dot(x2[...], W1[...], preferred_element_type=jnp.float32)
    h = jnp.maximum(h + b1[...], 0.0)
    h = jnp.dot(h, W2[...], preferred_element_type=jnp.float32)
    h = jnp.maximum(h + b2[...], 0.0)
    feat = jnp.dot(h, W3[...], preferred_element_type=jnp.float32) + b3[...]
    logits = jnp.dot(feat, cw[...], preferred_element_type=jnp.float32)
    logits = logits * g1[...] + bt1[...]
    mx = jnp.max(logits, axis=1, keepdims=True)
    ex = jnp.exp(logits - mx)
    act = ex / jnp.sum(ex, axis=1, keepdims=True)
    actm = act * maskv_ref[...]
    a_sum = jnp.sum(actm, axis=0, keepdims=True)
    vlad = lax.dot_general(
        feat, actm, (((0,), (0,)), ((), ())),
        preferred_element_type=jnp.float32,
    )
    vlad = vlad - a_sum * cw2[...]
    nrm = jnp.sqrt(jnp.sum(vlad * vlad, axis=0, keepdims=True))
    u = vlad / (nrm + 1e-12)
    g = jnp.sqrt(jnp.sum(u * u))
    u = u / (g + 1e-12)

    for d in range(D_OUT):
        uf_ref[0:1, d * CLUSTER:(d + 1) * CLUSTER] = u[d:d + 1, :]
    cp.wait()
    t = jnp.dot(uf_ref[...], hw_vmem[...], preferred_element_type=jnp.float32)
    o = t * g2[...] + b2h[...]
    z = jnp.dot(o, gw[...], preferred_element_type=jnp.float32)
    z = z * gg[...] + gb[...]
    gates = 1.0 / (1.0 + jnp.exp(-z))
    out_ref[...] = o * gates


def _fused(x2, Wq, Wk, Wv, W1, b1, W2, b2, W3, b3, cw, g1, bt1, cw2,
           hw, g2, b2h, gw, gg, gb):
    n_in = 20
    specs = [pl.BlockSpec(memory_space=pl.ANY) if i == 14
             else pl.BlockSpec() for i in range(n_in)]
    return pl.pallas_call(
        _fused_kernel,
        in_specs=specs,
        out_shape=(
            jax.ShapeDtypeStruct((N_BLKS, ROW_BLK), jnp.float32),
            jax.ShapeDtypeStruct((1, G_DIM), jnp.float32),
        ),
        scratch_shapes=[
            pltpu.VMEM((N_POINTS, 1), jnp.float32),
            pltpu.VMEM((1, D_OUT * CLUSTER), jnp.float32),
            pltpu.VMEM((D_OUT * CLUSTER, G_DIM), jnp.float32),
            pltpu.SemaphoreType.DMA,
        ],
    )(x2, Wq, Wk, Wv, W1, b1, W2, b2, W3, b3, cw, g1, bt1, cw2,
      hw, g2, b2h, gw, gg, gb)


def kernel(x, Wq, Wk, Wv, W1, b1, W2, b2, W3, b3, cluster_w, bn1_gamma,
           bn1_beta, cluster_w2, hidden_w, bn2_gamma, bn2_beta, gating_w,
           gbn_gamma, gbn_beta):
    x2 = x[0]
    xg, out = _fused(
        x2, Wq, Wk, Wv, W1, b1.reshape(1, -1), W2, b2.reshape(1, -1),
        W3, b3.reshape(1, -1), cluster_w, bn1_gamma.reshape(1, -1),
        bn1_beta.reshape(1, -1), cluster_w2[0], hidden_w,
        bn2_gamma.reshape(1, -1), bn2_beta.reshape(1, -1), gating_w,
        gbn_gamma.reshape(1, -1), gbn_beta.reshape(1, -1),
    )
    return out, xg.reshape(1, N_POINTS)
